# Initial kernel scaffold; baseline (speedup 1.0000x reference)
import jax
import jax.numpy as jnp
from jax.experimental import pallas as pl


def kernel(q, k, v, key_cache, value_cache, block_tables, seq_lens, positions):
    raise NotImplementedError("write your pallas kernel here")


# speedup vs baseline: 1.1132x; 1.1132x over previous
import math

import jax
import jax.numpy as jnp
from jax.experimental import pallas as pl
from jax.experimental.pallas import tpu as pltpu

B = 16
H = 8
D = 128
BS = 16
CTX = 1024
NUM_BLOCKS = 1024
MAXB = 64
KV_SCALE = 1.0
ROPE_BASE = 10000.0
HALF = D // 2
SCALE = 1.0 / math.sqrt(D)

CH = 8
T = CH * BS

_CONTRACT_MINOR = (((1,), (1,)), ((), ()))
_CONTRACT_MAJOR = (((0,), (0,)), ((), ()))


def _inv_freq_row():
  fidx = jax.lax.broadcasted_iota(jnp.int32, (1, HALF), 1).astype(jnp.float32)
  return 1.0 / (ROPE_BASE ** (fidx / HALF))


def _rot_coeffs(pos_f32_col):
  ang = pos_f32_col * _inv_freq_row()
  c = jnp.cos(ang)
  s = jnp.sin(ang)
  return jnp.concatenate([c, c], axis=-1), jnp.concatenate([-s, s], axis=-1)


def _swap_halves(x):
  return jnp.concatenate([x[..., HALF:], x[..., :HALF]], axis=-1)


def _attn_body(bt_ref, sl_ref, q_ref, k_ref, v_ref, kc_ref, vc_ref, o_ref,
               kbuf, vbuf, ksem, vsem):
  i = pl.program_id(0)

  s = 257 + sl_ref[i] % (2048 - 257)
  num_past = s - 1
  rem = num_past % BS
  within = num_past < CTX
  full = jnp.where(within, num_past // BS, (CTX // BS) - 1)
  n_valid = full * BS + rem
  nblocks = (n_valid + BS - 1) // BS
  nchunks = (nblocks + CH - 1) // CH

  def copies(c, slot):
    out = []
    for b in range(CH):
      safe = jnp.minimum(c * CH + b, nblocks - 1)
      bt = bt_ref[i, safe]
      out.append(pltpu.make_async_copy(
          kc_ref.at[bt], kbuf.at[slot, pl.ds(b * BS, BS)], ksem.at[slot]))
      out.append(pltpu.make_async_copy(
          vc_ref.at[bt], vbuf.at[slot, pl.ds(b * BS, BS)], vsem.at[slot]))
    return out

  def issue(c, slot):
    for cp in copies(c, slot):
      cp.start()

  def wait(c, slot):
    for cp in copies(c, slot):
      cp.wait()

  issue(0, 0)

  cur_pos = jnp.minimum(num_past, CTX - 1)
  qC, qS = _rot_coeffs(jnp.full((1, 1), cur_pos, jnp.float32))
  qh = q_ref[0]
  kh = k_ref[0]
  q_rot = qh * qC + _swap_halves(qh) * qS
  k_rot = kh * qC + _swap_halves(kh) * qS

  jt = jax.lax.broadcasted_iota(jnp.int32, (T, 1), 0)

  def chunk_body(c, carry):
    ms, ls, accs = carry
    slot = jax.lax.rem(c, 2)

    @pl.when(c + 1 < nchunks)
    def _():
      issue(c + 1, 1 - slot)

    wait(c, slot)

    j = c * T + jt
    mask = j < n_valid
    pos = jnp.where(within, j,
                    jnp.where(j < BS, j, j + BS - 1 - rem)).astype(jnp.float32)
    C, S = _rot_coeffs(pos)

    ms_n, ls_n, accs_n = [], [], []
    for h in range(H):
      Xh = kbuf[slot, :, h, :] * KV_SCALE
      Xr = Xh * C + _swap_halves(Xh) * S
      qr = q_rot[h:h + 1, :]
      sc = jax.lax.dot_general(Xr, qr, _CONTRACT_MINOR,
                               preferred_element_type=jnp.float32) * SCALE
      sc = jnp.where(mask, sc, -1e30)
      m_c = jnp.max(sc)
      m_new = jnp.maximum(ms[h], m_c)
      alpha = jnp.exp(ms[h] - m_new)
      p = jnp.exp(sc - m_new)
      p = jnp.where(mask, p, 0.0)
      l_new = alpha * ls[h] + jnp.sum(p)
      Vh = vbuf[slot, :, h, :] * KV_SCALE
      pv = jax.lax.dot_general(p, Vh, _CONTRACT_MAJOR,
                               preferred_element_type=jnp.float32)
      acc_new = alpha * accs[h] + pv
      ms_n.append(m_new)
      ls_n.append(l_new)
      accs_n.append(acc_new)
    return tuple(ms_n), tuple(ls_n), tuple(accs_n)

  neg = jnp.float32(-1e30)
  m0 = tuple(neg for _ in range(H))
  l0 = tuple(jnp.float32(0.0) for _ in range(H))
  a0 = tuple(jnp.zeros((1, D), jnp.float32) for _ in range(H))
  ms, ls, accs = jax.lax.fori_loop(0, nchunks, chunk_body, (m0, l0, a0))

  for h in range(H):
    s_cur = jnp.sum(q_rot[h:h + 1, :] * k_rot[h:h + 1, :]) * SCALE
    m_f = jnp.maximum(ms[h], s_cur)
    alpha = jnp.exp(ms[h] - m_f)
    p_cur = jnp.exp(s_cur - m_f)
    l_f = alpha * ls[h] + p_cur
    out_h = (alpha * accs[h] + p_cur * v_ref[0, h:h + 1, :]) / l_f
    o_ref[0, h:h + 1, :] = out_h


@jax.jit
def kernel(q, k, v, key_cache, value_cache, block_tables, seq_lens, positions):
  del positions
  grid_spec = pltpu.PrefetchScalarGridSpec(
      num_scalar_prefetch=2,
      grid=(B,),
      in_specs=[
          pl.BlockSpec((1, H, D), lambda i, bt, sl: (i, 0, 0)),
          pl.BlockSpec((1, H, D), lambda i, bt, sl: (i, 0, 0)),
          pl.BlockSpec((1, H, D), lambda i, bt, sl: (i, 0, 0)),
          pl.BlockSpec(memory_space=pl.MemorySpace.ANY),
          pl.BlockSpec(memory_space=pl.MemorySpace.ANY),
      ],
      out_specs=pl.BlockSpec((1, H, D), lambda i, bt, sl: (i, 0, 0)),
      scratch_shapes=[
          pltpu.VMEM((2, T, H, D), jnp.float32),
          pltpu.VMEM((2, T, H, D), jnp.float32),
          pltpu.SemaphoreType.DMA((2,)),
          pltpu.SemaphoreType.DMA((2,)),
      ],
  )
  out = pl.pallas_call(
      _attn_body,
      grid_spec=grid_spec,
      out_shape=jax.ShapeDtypeStruct((B, H, D), jnp.float32),
  )(block_tables, seq_lens, q.reshape(B, H, D), k.reshape(B, H, D),
    v.reshape(B, H, D), key_cache, value_cache)
  return out.reshape(B, H * D)
